# baseline (speedup 1.0000x reference)
import functools

import jax
import jax.numpy as jnp
from jax import lax
from jax.experimental import pallas as pl
from jax.experimental.pallas import tpu as pltpu
from jax.experimental.pallas import tpu_sc as plsc

B = 4096
QL = 20
DL = 200
E = 64
W = 128
NC = 2
NS = 16
NW = NC * NS
RPW = B // NW
CQ = 8
CD = 1
LANES = 16
EB = E // LANES


def _pool_body(q_hbm, d_hbm, table_hbm, qout_hbm, dout_hbm,
               qidx0, qidx1, didx0, didx1, qrows0, qrows1, drows0, drows1,
               qacc_v, dacc_v, sem0, sem1):
    wid = lax.axis_index("s") * NC + lax.axis_index("c")
    base = wid * RPW
    sems = (sem0, sem1)

    def phase(idx_hbm, L, rows_per_chunk, idxs, rowss, acc_v):
        nchunks = RPW // rows_per_chunk
        inv = 1.0 / L

        def start(c, b):
            r0 = base + c * rows_per_chunk
            pltpu.sync_copy(idx_hbm.at[pl.ds(r0, rows_per_chunk)], idxs[b])
            for rr in range(rows_per_chunk):
                pltpu.async_copy(table_hbm.at[idxs[b].at[rr]],
                                 rowss[b].at[pl.ds(rr * L, L)], sems[b])

        def wait(b):
            for rr in range(rows_per_chunk):
                pltpu.make_async_copy(table_hbm.at[idxs[b].at[rr]],
                                      rowss[b].at[pl.ds(rr * L, L)],
                                      sems[b]).wait()

        def accum(c, b):
            rows_v = rowss[b]
            for rr in range(rows_per_chunk):
                def seq_body(j, accs):
                    row = rr * L + j
                    return tuple(accs[e] + rows_v[row, pl.ds(e * LANES, LANES)]
                                 for e in range(EB))
                accs = lax.fori_loop(
                    0, L, seq_body,
                    tuple(jnp.zeros((LANES,), jnp.float32) for _ in range(EB)),
                    unroll=4)
                out_row = c * rows_per_chunk + rr
                for e in range(EB):
                    acc_v[out_row, pl.ds(e * LANES, LANES)] = accs[e] * inv

        start(0, 0)

        @pl.loop(0, nchunks // 2)
        def _pair(p):
            c0 = 2 * p
            start(c0 + 1, 1)
            wait(0)
            accum(c0, 0)

            @pl.when(c0 + 2 < nchunks)
            def _prefetch():
                start(c0 + 2, 0)

            wait(1)
            accum(c0 + 1, 1)

    phase(q_hbm, QL, CQ, (qidx0, qidx1), (qrows0, qrows1), qacc_v)
    phase(d_hbm, DL, CD, (didx0, didx1), (drows0, drows1), dacc_v)
    pltpu.sync_copy(qacc_v, qout_hbm.at[pl.ds(base, RPW)])
    pltpu.sync_copy(dacc_v, dout_hbm.at[pl.ds(base, RPW)])


V = 1000000
FB = V // W
TAIL = V - FB * W
BASE_BLK = FB // NW
EXTRA = FB - BASE_BLK * NW


def _transpose_body(tt_hbm, out_hbm, in0, in1, ot0, ot1, tin, tout,
                    is0, is1, os0, os1):
    wid = lax.axis_index("s") * NC + lax.axis_index("c")
    nb = BASE_BLK + jnp.where(wid < EXTRA, 1, 0)
    sb = wid * BASE_BLK + jnp.minimum(wid, EXTRA)
    ins = (in0, in1)
    outs = (ot0, ot1)
    isems = (is0, is1)
    osems = (os0, os1)

    def start_in(c, b):
        pltpu.async_copy(tt_hbm.at[:, pl.ds(c * W, W)], ins[b], isems[b])

    def wait_in(b):
        pltpu.make_async_copy(tt_hbm.at[:, pl.ds(0, W)], ins[b], isems[b]).wait()

    def start_out(c, b):
        pltpu.async_copy(outs[b], out_hbm.at[pl.ds(c * W, W)], osems[b])

    def wait_out(b):
        pltpu.make_async_copy(outs[b], out_hbm.at[pl.ds(0, W)], osems[b]).wait()

    def transpose_block(b):
        src = ins[b]
        dst = outs[b]
        lanes = lax.iota(jnp.int32, LANES)

        @pl.loop(0, W, unroll=4)
        def _tok(v):
            vcol = jnp.full((LANES,), 0, jnp.int32) + v
            for k in range(EB):
                vals = plsc.load_gather(src, [k * LANES + lanes, vcol])
                dst[v, pl.ds(k * LANES, LANES)] = vals

    def do_pair(p):
        c0 = sb + 2 * p

        @pl.when(2 * p < nb)
        def _even():
            wait_in(0)

            @pl.when(2 * p + 1 < nb)
            def _pre1():
                start_in(c0 + 1, 1)

            @pl.when(p > 0)
            def _drain0():
                wait_out(0)

            transpose_block(0)
            start_out(c0, 0)

        @pl.when(2 * p + 1 < nb)
        def _odd():
            wait_in(1)

            @pl.when(2 * p + 2 < nb)
            def _pre0():
                start_in(c0 + 2, 0)

            @pl.when(p > 0)
            def _drain1():
                wait_out(1)

            transpose_block(1)
            start_out(c0 + 1, 1)

    start_in(sb, 0)
    npairs = (BASE_BLK + 2) // 2

    @pl.loop(0, npairs)
    def _pairs(p):
        do_pair(p)

    @pl.when(nb >= 1)
    def _d0():
        wait_out(0)

    @pl.when(nb >= 2)
    def _d1():
        wait_out(1)

    @pl.when(wid == NW - 1)
    def _tail():
        pltpu.sync_copy(tt_hbm.at[:, pl.ds(FB * W, TAIL)], tin)
        lanes = lax.iota(jnp.int32, LANES)

        @pl.loop(0, TAIL)
        def _tok(v):
            vcol = jnp.full((LANES,), 0, jnp.int32) + v
            for k in range(EB):
                vals = plsc.load_gather(tin, [k * LANES + lanes, vcol])
                tout[v, pl.ds(k * LANES, LANES)] = vals

        pltpu.sync_copy(tout, out_hbm.at[pl.ds(FB * W, TAIL)])


@functools.lru_cache(maxsize=None)
def _transpose_kernel():
    return functools.partial(
        pl.kernel,
        out_type=jax.ShapeDtypeStruct((V, W), jnp.float32),
        mesh=plsc.VectorSubcoreMesh(core_axis_name="c", subcore_axis_name="s",
                                    num_cores=NC, num_subcores=NS),
        scratch_types=[
            pltpu.VMEM((E, W), jnp.float32),
            pltpu.VMEM((E, W), jnp.float32),
            pltpu.VMEM((W, W), jnp.float32),
            pltpu.VMEM((W, W), jnp.float32),
            pltpu.VMEM((E, TAIL), jnp.float32),
            pltpu.VMEM((TAIL, W), jnp.float32),
            pltpu.SemaphoreType.DMA,
            pltpu.SemaphoreType.DMA,
            pltpu.SemaphoreType.DMA,
            pltpu.SemaphoreType.DMA,
        ],
        compiler_params=pltpu.CompilerParams(use_tc_tiling_on_sc=True,
                                             needs_layout_passes=False),
    )(_transpose_body)


@functools.lru_cache(maxsize=None)
def _pool_kernel():
    return functools.partial(
        pl.kernel,
        out_type=(jax.ShapeDtypeStruct((B, E), jnp.float32),
                  jax.ShapeDtypeStruct((B, E), jnp.float32)),
        mesh=plsc.VectorSubcoreMesh(core_axis_name="c", subcore_axis_name="s",
                                    num_cores=NC, num_subcores=NS),
        scratch_types=[
            pltpu.VMEM((CQ, QL), jnp.int32),
            pltpu.VMEM((CQ, QL), jnp.int32),
            pltpu.VMEM((CD, DL), jnp.int32),
            pltpu.VMEM((CD, DL), jnp.int32),
            pltpu.VMEM((CQ * QL, W), jnp.float32),
            pltpu.VMEM((CQ * QL, W), jnp.float32),
            pltpu.VMEM((CD * DL, W), jnp.float32),
            pltpu.VMEM((CD * DL, W), jnp.float32),
            pltpu.VMEM((RPW, E), jnp.float32),
            pltpu.VMEM((RPW, E), jnp.float32),
            pltpu.SemaphoreType.DMA,
            pltpu.SemaphoreType.DMA,
        ],
        compiler_params=pltpu.CompilerParams(use_tc_tiling_on_sc=False),
    )(_pool_body)


def _head_body(q_ref, d_ref, wq0, bq0, wqv, bqv, wd0, bd0, wdv, bdv, wo, bo,
               out_ref, cos_ref):
    fq = jnp.dot(wq0[...], wqv[...], preferred_element_type=jnp.float32)
    bq = jnp.dot(bq0[...], wqv[...], preferred_element_type=jnp.float32) + bqv[...]
    fd = jnp.dot(wd0[...], wdv[...], preferred_element_type=jnp.float32)
    bd = jnp.dot(bd0[...], wdv[...], preferred_element_type=jnp.float32) + bdv[...]
    qv = jnp.dot(q_ref[...], fq, preferred_element_type=jnp.float32) + bq
    dv = jnp.dot(d_ref[...], fd, preferred_element_type=jnp.float32) + bd
    qn = qv / jnp.sqrt(jnp.maximum(jnp.sum(qv * qv, axis=-1, keepdims=True), 1e-12))
    dn = dv / jnp.sqrt(jnp.maximum(jnp.sum(dv * dv, axis=-1, keepdims=True), 1e-12))
    cos = jnp.sum(qn * dn, axis=-1, keepdims=True)
    cos_ref[...] = cos
    out_ref[...] = jax.nn.sigmoid(cos * wo[0, 0] + bo[0, 0])


_head_call = pl.pallas_call(
    _head_body,
    out_shape=(jax.ShapeDtypeStruct((B, 1), jnp.float32),
               jax.ShapeDtypeStruct((B, 1), jnp.float32)),
)


def kernel(query, doc, table, Wq0, bq0, Wqv, bqv, Wd0, bd0, Wdv, bdv, Wo, bo):
    table_pad = _transpose_kernel()(table.T)
    q_emb, d_emb = _pool_kernel()(query, doc, table_pad)
    out, cos = _head_call(q_emb, d_emb,
                          Wq0, bq0.reshape(1, -1), Wqv, bqv.reshape(1, -1),
                          Wd0, bd0.reshape(1, -1), Wdv, bdv.reshape(1, -1),
                          Wo, bo.reshape(1, 1))
    return (out, cos)


# speedup vs baseline: 1.3121x; 1.3121x over previous
import functools

import jax
import jax.numpy as jnp
from jax import lax
from jax.experimental import pallas as pl
from jax.experimental.pallas import tpu as pltpu
from jax.experimental.pallas import tpu_sc as plsc

B = 4096
QL = 20
DL = 200
E = 64
W = 128
NC = 2
NS = 16
NW = NC * NS
RPW = B // NW
CQ = 8
CD = 1
LANES = 16
EB = E // LANES


def _pool_body(q_hbm, d_hbm, table_hbm, qout_hbm, dout_hbm,
               qidx0, qidx1, didx0, didx1, qrows0, qrows1, drows0, drows1,
               qacc_v, dacc_v, sem0, sem1):
    wid = lax.axis_index("s") * NC + lax.axis_index("c")
    base = wid * RPW
    sems = (sem0, sem1)

    def phase(idx_hbm, L, rows_per_chunk, idxs, rowss, acc_v):
        nchunks = RPW // rows_per_chunk
        inv = 1.0 / L

        def start(c, b):
            r0 = base + c * rows_per_chunk
            pltpu.sync_copy(idx_hbm.at[pl.ds(r0, rows_per_chunk)], idxs[b])
            for rr in range(rows_per_chunk):
                pltpu.async_copy(table_hbm.at[idxs[b].at[rr]],
                                 rowss[b].at[pl.ds(rr * L, L)], sems[b])

        def wait(b):
            for rr in range(rows_per_chunk):
                pltpu.make_async_copy(table_hbm.at[idxs[b].at[rr]],
                                      rowss[b].at[pl.ds(rr * L, L)],
                                      sems[b]).wait()

        def accum(c, b):
            rows_v = rowss[b]
            for rr in range(rows_per_chunk):
                def seq_body(j, accs):
                    row = rr * L + j
                    return tuple(accs[e] + rows_v[row, pl.ds(e * LANES, LANES)]
                                 for e in range(EB))
                accs = lax.fori_loop(
                    0, L, seq_body,
                    tuple(jnp.zeros((LANES,), jnp.float32) for _ in range(EB)),
                    unroll=4)
                out_row = c * rows_per_chunk + rr
                for e in range(EB):
                    acc_v[out_row, pl.ds(e * LANES, LANES)] = accs[e] * inv

        start(0, 0)

        @pl.loop(0, nchunks // 2)
        def _pair(p):
            c0 = 2 * p
            start(c0 + 1, 1)
            wait(0)
            accum(c0, 0)

            @pl.when(c0 + 2 < nchunks)
            def _prefetch():
                start(c0 + 2, 0)

            wait(1)
            accum(c0 + 1, 1)

    phase(q_hbm, QL, CQ, (qidx0, qidx1), (qrows0, qrows1), qacc_v)
    phase(d_hbm, DL, CD, (didx0, didx1), (drows0, drows1), dacc_v)
    pltpu.sync_copy(qacc_v, qout_hbm.at[pl.ds(base, RPW)])
    pltpu.sync_copy(dacc_v, dout_hbm.at[pl.ds(base, RPW)])


V = 1000000
FB = V // W
TAIL = V - FB * W
BASE_BLK = FB // NW
EXTRA = FB - BASE_BLK * NW


def _transpose_body(tt_hbm, out_hbm, in0, in1, ot0, ot1, tin, tout,
                    is0, is1, os0, os1):
    wid = lax.axis_index("s") * NC + lax.axis_index("c")
    nb = BASE_BLK + jnp.where(wid < EXTRA, 1, 0)
    sb = wid * BASE_BLK + jnp.minimum(wid, EXTRA)
    ins = (in0, in1)
    outs = (ot0, ot1)
    isems = (is0, is1)
    osems = (os0, os1)

    def start_in(c, b):
        pltpu.async_copy(tt_hbm.at[:, pl.ds(c * W, W)], ins[b], isems[b])

    def wait_in(b):
        pltpu.make_async_copy(tt_hbm.at[:, pl.ds(0, W)], ins[b], isems[b]).wait()

    def start_out(c, b):
        pltpu.async_copy(outs[b], out_hbm.at[pl.ds(c * W, W)], osems[b])

    def wait_out(b):
        pltpu.make_async_copy(outs[b], out_hbm.at[pl.ds(0, W)], osems[b]).wait()

    def transpose_block(b):
        src = ins[b]
        dst = outs[b]
        lanes = lax.iota(jnp.int32, LANES)

        TG = 4

        @pl.loop(0, W // TG)
        def _tok(g):
            v0 = g * TG
            vals = []
            for t in range(TG):
                vcol = jnp.full((LANES,), 0, jnp.int32) + (v0 + t)
                for k in range(EB):
                    vals.append(plsc.load_gather(src, [k * LANES + lanes, vcol]))
            for t in range(TG):
                for k in range(EB):
                    dst[v0 + t, pl.ds(k * LANES, LANES)] = vals[t * EB + k]

    def do_pair(p):
        c0 = sb + 2 * p

        @pl.when(2 * p < nb)
        def _even():
            wait_in(0)

            @pl.when(2 * p + 1 < nb)
            def _pre1():
                start_in(c0 + 1, 1)

            @pl.when(p > 0)
            def _drain0():
                wait_out(0)

            transpose_block(0)
            start_out(c0, 0)

        @pl.when(2 * p + 1 < nb)
        def _odd():
            wait_in(1)

            @pl.when(2 * p + 2 < nb)
            def _pre0():
                start_in(c0 + 2, 0)

            @pl.when(p > 0)
            def _drain1():
                wait_out(1)

            transpose_block(1)
            start_out(c0 + 1, 1)

    start_in(sb, 0)
    npairs = (BASE_BLK + 2) // 2

    @pl.loop(0, npairs)
    def _pairs(p):
        do_pair(p)

    @pl.when(nb >= 1)
    def _d0():
        wait_out(0)

    @pl.when(nb >= 2)
    def _d1():
        wait_out(1)

    @pl.when(wid == NW - 1)
    def _tail():
        pltpu.sync_copy(tt_hbm.at[:, pl.ds(FB * W, TAIL)], tin)
        lanes = lax.iota(jnp.int32, LANES)

        @pl.loop(0, TAIL // 4)
        def _tok(g):
            v0 = g * 4
            vals = []
            for t in range(4):
                vcol = jnp.full((LANES,), 0, jnp.int32) + (v0 + t)
                for k in range(EB):
                    vals.append(plsc.load_gather(tin, [k * LANES + lanes, vcol]))
            for t in range(4):
                for k in range(EB):
                    tout[v0 + t, pl.ds(k * LANES, LANES)] = vals[t * EB + k]

        pltpu.sync_copy(tout, out_hbm.at[pl.ds(FB * W, TAIL)])


@functools.lru_cache(maxsize=None)
def _transpose_kernel():
    return functools.partial(
        pl.kernel,
        out_type=jax.ShapeDtypeStruct((V, W), jnp.float32),
        mesh=plsc.VectorSubcoreMesh(core_axis_name="c", subcore_axis_name="s",
                                    num_cores=NC, num_subcores=NS),
        scratch_types=[
            pltpu.VMEM((E, W), jnp.float32),
            pltpu.VMEM((E, W), jnp.float32),
            pltpu.VMEM((W, W), jnp.float32),
            pltpu.VMEM((W, W), jnp.float32),
            pltpu.VMEM((E, TAIL), jnp.float32),
            pltpu.VMEM((TAIL, W), jnp.float32),
            pltpu.SemaphoreType.DMA,
            pltpu.SemaphoreType.DMA,
            pltpu.SemaphoreType.DMA,
            pltpu.SemaphoreType.DMA,
        ],
        compiler_params=pltpu.CompilerParams(use_tc_tiling_on_sc=True,
                                             needs_layout_passes=False),
    )(_transpose_body)


@functools.lru_cache(maxsize=None)
def _pool_kernel():
    return functools.partial(
        pl.kernel,
        out_type=(jax.ShapeDtypeStruct((B, E), jnp.float32),
                  jax.ShapeDtypeStruct((B, E), jnp.float32)),
        mesh=plsc.VectorSubcoreMesh(core_axis_name="c", subcore_axis_name="s",
                                    num_cores=NC, num_subcores=NS),
        scratch_types=[
            pltpu.VMEM((CQ, QL), jnp.int32),
            pltpu.VMEM((CQ, QL), jnp.int32),
            pltpu.VMEM((CD, DL), jnp.int32),
            pltpu.VMEM((CD, DL), jnp.int32),
            pltpu.VMEM((CQ * QL, W), jnp.float32),
            pltpu.VMEM((CQ * QL, W), jnp.float32),
            pltpu.VMEM((CD * DL, W), jnp.float32),
            pltpu.VMEM((CD * DL, W), jnp.float32),
            pltpu.VMEM((RPW, E), jnp.float32),
            pltpu.VMEM((RPW, E), jnp.float32),
            pltpu.SemaphoreType.DMA,
            pltpu.SemaphoreType.DMA,
        ],
        compiler_params=pltpu.CompilerParams(use_tc_tiling_on_sc=False),
    )(_pool_body)


def _head_body(q_ref, d_ref, wq0, bq0, wqv, bqv, wd0, bd0, wdv, bdv, wo, bo,
               out_ref, cos_ref):
    fq = jnp.dot(wq0[...], wqv[...], preferred_element_type=jnp.float32)
    bq = jnp.dot(bq0[...], wqv[...], preferred_element_type=jnp.float32) + bqv[...]
    fd = jnp.dot(wd0[...], wdv[...], preferred_element_type=jnp.float32)
    bd = jnp.dot(bd0[...], wdv[...], preferred_element_type=jnp.float32) + bdv[...]
    qv = jnp.dot(q_ref[...], fq, preferred_element_type=jnp.float32) + bq
    dv = jnp.dot(d_ref[...], fd, preferred_element_type=jnp.float32) + bd
    qn = qv / jnp.sqrt(jnp.maximum(jnp.sum(qv * qv, axis=-1, keepdims=True), 1e-12))
    dn = dv / jnp.sqrt(jnp.maximum(jnp.sum(dv * dv, axis=-1, keepdims=True), 1e-12))
    cos = jnp.sum(qn * dn, axis=-1, keepdims=True)
    cos_ref[...] = cos
    out_ref[...] = jax.nn.sigmoid(cos * wo[0, 0] + bo[0, 0])


_head_call = pl.pallas_call(
    _head_body,
    out_shape=(jax.ShapeDtypeStruct((B, 1), jnp.float32),
               jax.ShapeDtypeStruct((B, 1), jnp.float32)),
)


def kernel(query, doc, table, Wq0, bq0, Wqv, bqv, Wd0, bd0, Wdv, bdv, Wo, bo):
    table_pad = _transpose_kernel()(table.T)
    q_emb, d_emb = _pool_kernel()(query, doc, table_pad)
    out, cos = _head_call(q_emb, d_emb,
                          Wq0, bq0.reshape(1, -1), Wqv, bqv.reshape(1, -1),
                          Wd0, bd0.reshape(1, -1), Wdv, bdv.reshape(1, -1),
                          Wo, bo.reshape(1, 1))
    return (out, cos)


# speedup vs baseline: 2.2976x; 1.7511x over previous
import functools

import jax
import jax.numpy as jnp
from jax import lax
from jax.experimental import pallas as pl
from jax.experimental.pallas import tpu as pltpu
from jax.experimental.pallas import tpu_sc as plsc

B = 4096
QL = 20
DL = 200
E = 64
NC = 2
NS = 16
NW = NC * NS
RPW = B // NW
CQ = 16
CD = 2
LANES = 16
EB = E // LANES


def _pool_body(q_hbm, d_hbm, table_hbm, qout_hbm, dout_hbm,
               qidx0, qidx1, didx0, didx1, qrows0, qrows1, drows0, drows1,
               qacc_v, dacc_v, sem0, sem1):
    wid = lax.axis_index("s") * NC + lax.axis_index("c")
    base = wid * RPW
    sems = (sem0, sem1)

    def phase(idx_hbm, L, rows_per_chunk, idxs, rowss, acc_v):
        nchunks = RPW // rows_per_chunk
        n = rows_per_chunk * L
        inv = 1.0 / L

        def start(c, b):
            r0 = base + c * rows_per_chunk
            pltpu.sync_copy(idx_hbm.at[pl.ds(r0 * L, n)], idxs[b])
            pltpu.async_copy(table_hbm.at[idxs[b]], rowss[b], sems[b])

        def wait(b):
            pltpu.make_async_copy(table_hbm.at[idxs[b]], rowss[b], sems[b]).wait()

        def accum(c, b):
            rows_v = rowss[b]
            for rr in range(rows_per_chunk):
                def seq_body(j, accs):
                    row = rr * L + j
                    return tuple(accs[e] + rows_v[row, pl.ds(e * LANES, LANES)]
                                 for e in range(EB))
                accs = lax.fori_loop(
                    0, L, seq_body,
                    tuple(jnp.zeros((LANES,), jnp.float32) for _ in range(EB)),
                    unroll=4)
                out_row = c * rows_per_chunk + rr
                for e in range(EB):
                    acc_v[out_row, pl.ds(e * LANES, LANES)] = accs[e] * inv

        start(0, 0)

        @pl.loop(0, nchunks // 2)
        def _pair(p):
            c0 = 2 * p
            start(c0 + 1, 1)
            wait(0)
            accum(c0, 0)

            @pl.when(c0 + 2 < nchunks)
            def _prefetch():
                start(c0 + 2, 0)

            wait(1)
            accum(c0 + 1, 1)

    phase(q_hbm, QL, CQ, (qidx0, qidx1), (qrows0, qrows1), qacc_v)
    phase(d_hbm, DL, CD, (didx0, didx1), (drows0, drows1), dacc_v)
    pltpu.sync_copy(qacc_v, qout_hbm.at[pl.ds(base, RPW)])
    pltpu.sync_copy(dacc_v, dout_hbm.at[pl.ds(base, RPW)])


@functools.lru_cache(maxsize=None)
def _pool_kernel():
    return functools.partial(
        pl.kernel,
        out_type=(jax.ShapeDtypeStruct((B, E), jnp.float32),
                  jax.ShapeDtypeStruct((B, E), jnp.float32)),
        mesh=plsc.VectorSubcoreMesh(core_axis_name="c", subcore_axis_name="s",
                                    num_cores=NC, num_subcores=NS),
        scratch_types=[
            pltpu.VMEM((CQ * QL,), jnp.int32),
            pltpu.VMEM((CQ * QL,), jnp.int32),
            pltpu.VMEM((CD * DL,), jnp.int32),
            pltpu.VMEM((CD * DL,), jnp.int32),
            pltpu.VMEM((CQ * QL, E), jnp.float32),
            pltpu.VMEM((CQ * QL, E), jnp.float32),
            pltpu.VMEM((CD * DL, E), jnp.float32),
            pltpu.VMEM((CD * DL, E), jnp.float32),
            pltpu.VMEM((RPW, E), jnp.float32),
            pltpu.VMEM((RPW, E), jnp.float32),
            pltpu.SemaphoreType.DMA,
            pltpu.SemaphoreType.DMA,
        ],
        compiler_params=pltpu.CompilerParams(use_tc_tiling_on_sc=False),
    )(_pool_body)


def _head_body(q_ref, d_ref, wq0, bq0, wqv, bqv, wd0, bd0, wdv, bdv, wo, bo,
               out_ref, cos_ref):
    fq = jnp.dot(wq0[...], wqv[...], preferred_element_type=jnp.float32)
    bq = jnp.dot(bq0[...], wqv[...], preferred_element_type=jnp.float32) + bqv[...]
    fd = jnp.dot(wd0[...], wdv[...], preferred_element_type=jnp.float32)
    bd = jnp.dot(bd0[...], wdv[...], preferred_element_type=jnp.float32) + bdv[...]
    qv = jnp.dot(q_ref[...], fq, preferred_element_type=jnp.float32) + bq
    dv = jnp.dot(d_ref[...], fd, preferred_element_type=jnp.float32) + bd
    qn = qv / jnp.sqrt(jnp.maximum(jnp.sum(qv * qv, axis=-1, keepdims=True), 1e-12))
    dn = dv / jnp.sqrt(jnp.maximum(jnp.sum(dv * dv, axis=-1, keepdims=True), 1e-12))
    cos = jnp.sum(qn * dn, axis=-1, keepdims=True)
    cos_ref[...] = cos
    out_ref[...] = jax.nn.sigmoid(cos * wo[0, 0] + bo[0, 0])


_head_call = pl.pallas_call(
    _head_body,
    out_shape=(jax.ShapeDtypeStruct((B, 1), jnp.float32),
               jax.ShapeDtypeStruct((B, 1), jnp.float32)),
)


def kernel(query, doc, table, Wq0, bq0, Wqv, bqv, Wd0, bd0, Wdv, bdv, Wo, bo):
    q_emb, d_emb = _pool_kernel()(query.reshape(-1), doc.reshape(-1), table)
    out, cos = _head_call(q_emb, d_emb,
                          Wq0, bq0.reshape(1, -1), Wqv, bqv.reshape(1, -1),
                          Wd0, bd0.reshape(1, -1), Wdv, bdv.reshape(1, -1),
                          Wo, bo.reshape(1, 1))
    return (out, cos)
